# Initial kernel scaffold; baseline (speedup 1.0000x reference)
import jax
import jax.numpy as jnp
from jax.experimental import pallas as pl


def kernel(x, edge_index, W1, a_src1, a_dst1, b1, W2, a_src2, a_dst2, b2):
    raise NotImplementedError("write your pallas kernel here")


# speedup vs baseline: 51.0480x; 51.0480x over previous
import functools

import jax
import jax.numpy as jnp
from jax import lax
from jax.experimental import pallas as pl
from jax.experimental.pallas import tpu as pltpu
from jax.experimental.pallas import tpu_sc as plsc

NSC = 2
NSUB = 16
LANES = 16
EB = 80


def _pad_rows(n):
    q = NSUB * 8
    return (n + q - 1) // q * q


def _tc1_body(heads, hid, x_ref, w_ref, waux_ref, h_ref, aux_ref):
    x = x_ref[...]
    h = jnp.dot(x, w_ref[...], preferred_element_type=jnp.float32)
    n = x.shape[0]
    for q in range(heads):
        h_ref[pl.ds(q * n, n), :] = h[:, q * hid:(q + 1) * hid]
    aux_ref[...] = lax.dot_general(
        waux_ref[...], x, (((1,), (1,)), ((), ())),
        preferred_element_type=jnp.float32)


def _tc1(x, w1, waux, heads, hid):
    n, _ = x.shape
    return pl.pallas_call(
        functools.partial(_tc1_body, heads, hid),
        out_shape=[
            jax.ShapeDtypeStruct((heads * n, hid), jnp.float32),
            jax.ShapeDtypeStruct((waux.shape[0], n), jnp.float32),
        ],
    )(x, w1, waux)


def _tc2_body(accm_ref, accd_ref, w2_ref, waux2_ref, b1_ref,
              h2_ref, aux2_ref):
    zs = []
    for q in range(4):
        num = accm_ref[q // 2, q % 2]
        den = accd_ref[q // 2, q % 2][:, 0:1]
        zs.append(num / (den + 1e-16))
    z = jnp.concatenate(zs, axis=1) + b1_ref[...]
    z = jnp.where(z > 0.0, z, jnp.exp(z) - 1.0)
    h2_ref[...] = jnp.dot(z, w2_ref[...], preferred_element_type=jnp.float32)
    aux2_ref[...] = lax.dot_general(
        z, waux2_ref[...], (((1,), (1,)), ((), ())),
        preferred_element_type=jnp.float32)


def _tc2(accm1, accd1, w2, waux2, b1, hid):
    np1 = accm1.shape[2]
    d1 = w2.shape[0]
    d2 = w2.shape[1]
    nblk = 8
    br = np1 // nblk
    return pl.pallas_call(
        _tc2_body,
        grid=(nblk,),
        in_specs=[
            pl.BlockSpec((2, 2, br, hid), lambda i: (0, 0, i, 0)),
            pl.BlockSpec((2, 2, br, LANES), lambda i: (0, 0, i, 0)),
            pl.BlockSpec((d1, d2), lambda i: (0, 0)),
            pl.BlockSpec((2, d1), lambda i: (0, 0)),
            pl.BlockSpec((1, d1), lambda i: (0, 0)),
        ],
        out_specs=[
            pl.BlockSpec((br, d2), lambda i: (i, 0)),
            pl.BlockSpec((br, 2), lambda i: (i, 0)),
        ],
        out_shape=[
            jax.ShapeDtypeStruct((np1, d2), jnp.float32),
            jax.ShapeDtypeStruct((np1, 2), jnp.float32),
        ],
    )(accm1, accd1, w2, waux2, b1)


def _tc3_body(accm_ref, accd_ref, b2_ref, out_ref):
    num = accm_ref[0] + accm_ref[1]
    den = accd_ref[0][:, 0:1] + accd_ref[1][:, 0:1]
    out_ref[...] = num / (den + 1e-16) + b2_ref[...]


def _tc3(accm2, accd2, b2):
    np1 = accm2.shape[1]
    d2 = b2.shape[1]
    nblk = 8
    br = np1 // nblk
    return pl.pallas_call(
        _tc3_body,
        grid=(nblk,),
        in_specs=[
            pl.BlockSpec((2, br, d2), lambda i: (0, i, 0)),
            pl.BlockSpec((2, br, LANES), lambda i: (0, i, 0)),
            pl.BlockSpec((1, d2), lambda i: (0, 0)),
        ],
        out_specs=pl.BlockSpec((br, d2), lambda i: (i, 0)),
        out_shape=jax.ShapeDtypeStruct((np1, d2), jnp.float32),
    )(accm2, accd2, b2)


def _make_sc1(n, e, hid, heads):
    np1 = _pad_rows(n)
    ept = e // NSUB
    ch = 4000
    nbc = ch // EB
    nch = ept // ch
    nrows = np1 // NSUB
    nseg = hid // LANES
    ngrp = EB // LANES
    npass = heads // NSC
    mesh = plsc.VectorSubcoreMesh(core_axis_name="c", subcore_axis_name="s")

    @functools.partial(
        pl.kernel,
        out_type=[
            jax.ShapeDtypeStruct((npass, NSC, np1, hid), jnp.float32),
            jax.ShapeDtypeStruct((npass, NSC, np1, LANES), jnp.float32),
        ],
        mesh=mesh,
        compiler_params=pltpu.CompilerParams(
            needs_layout_passes=False, use_tc_tiling_on_sc=False),
        scratch_types=[
            pltpu.VMEM((n,), jnp.float32),
            pltpu.VMEM((n,), jnp.float32),
            pltpu.VMEM((ch,), jnp.int32),
            pltpu.VMEM((ch,), jnp.int32),
            pltpu.VMEM((EB,), jnp.int32),
            pltpu.VMEM((EB,), jnp.int32),
            pltpu.VMEM((EB,), jnp.int32),
            pltpu.VMEM((EB,), jnp.int32),
            pltpu.VMEM((2, EB, hid), jnp.float32),
            pltpu.VMEM((2, EB, hid), jnp.float32),
            pltpu.VMEM((2, EB, LANES), jnp.float32),
            pltpu.VMEM_SHARED((np1, hid), jnp.float32),
            pltpu.VMEM_SHARED((np1, LANES), jnp.float32),
            pltpu.SemaphoreType.DMA,
            pltpu.SemaphoreType.DMA,
            pltpu.SemaphoreType.DMA,
            pltpu.SemaphoreType.DMA,
            pltpu.SemaphoreType.DMA,
            pltpu.SemaphoreType.DMA,
        ],
    )
    def sc1(h_ref, aux_ref,
            src_ref, dst_ref, outm_ref, outd_ref,
            asl, adl, srcl, dstl, gidx0, gidx1, sidx0, sidx1,
            gbuf, msg, dbuf, accm, accd,
            gsem0, gsem1, smsem0, smsem1, sdsem0, sdsem1):
        c = lax.axis_index("c")
        s = lax.axis_index("s")
        base = s * ept
        zvec = jnp.zeros((LANES,), jnp.float32)
        rchunks = [(k * EB, min(EB, nrows - k * EB))
                   for k in range((nrows + EB - 1) // EB)]

        def zero_acc():
            for r in range(EB):
                for k2 in range(hid // LANES):
                    msg[0, r, pl.ds(k2 * LANES, LANES)] = zvec
                dbuf[0, r, :] = zvec
            for (r0, rn) in rchunks:
                pltpu.sync_copy(msg.at[0, pl.ds(0, rn)],
                                accm.at[pl.ds(s * nrows + r0, rn)])
                pltpu.sync_copy(dbuf.at[0, pl.ds(0, rn)],
                                accd.at[pl.ds(s * nrows + r0, rn)])

        def flush_acc(outm_slice, outd_slice):
            for (r0, rn) in rchunks:
                pltpu.sync_copy(accm.at[pl.ds(s * nrows + r0, rn)],
                                msg.at[0, pl.ds(0, rn)])
                pltpu.sync_copy(msg.at[0, pl.ds(0, rn)],
                                outm_slice.at[pl.ds(s * nrows + r0, rn)])
                pltpu.sync_copy(accd.at[pl.ds(s * nrows + r0, rn)],
                                dbuf.at[0, pl.ds(0, rn)])
                pltpu.sync_copy(dbuf.at[0, pl.ds(0, rn)],
                                outd_slice.at[pl.ds(s * nrows + r0, rn)])

        def make_build(qoff):
            def build(b, slot):
                gidx = gidx0 if slot == 0 else gidx1
                for g in range(ngrp):
                    sv = srcl[pl.ds(b * EB + g * LANES, LANES)]
                    gidx[pl.ds(g * LANES, LANES)] = sv + qoff
                pltpu.async_copy(h_ref.at[gidx], gbuf.at[slot],
                                 gsem0 if slot == 0 else gsem1)
            return build

        def compute(b, slot):
            off = b * EB
            gs = gsem0 if slot == 0 else gsem1
            sms = smsem0 if slot == 0 else smsem1
            sds = sdsem0 if slot == 0 else sdsem1
            gidx = gidx0 if slot == 0 else gidx1
            sidx = sidx0 if slot == 0 else sidx1
            pltpu.make_async_copy(h_ref.at[gidx], gbuf.at[slot], gs).wait()
            for g in range(ngrp):
                sv = srcl[pl.ds(off + g * LANES, LANES)]
                dv = dstl[pl.ds(off + g * LANES, LANES)]
                sidx[pl.ds(g * LANES, LANES)] = dv
                a0 = plsc.load_gather(asl, [sv]) + plsc.load_gather(adl, [dv])
                a0 = jnp.maximum(a0, 0.2 * a0)
                w0v = jnp.exp(a0)
                for j in range(LANES):
                    ei = g * LANES + j
                    w0 = w0v[j]
                    for k in range(nseg):
                        msg[slot, ei, pl.ds(k * LANES, LANES)] = (
                            gbuf[slot, ei, pl.ds(k * LANES, LANES)] * w0)
                    dbuf[slot, ei, :] = jnp.full((LANES,), w0, jnp.float32)
            pltpu.sync_copy(msg.at[slot], accm.at[sidx], add=True)
            pltpu.sync_copy(dbuf.at[slot], accd.at[sidx], add=True)

        def drain(slot):
            del slot

        for p in range(npass):
            q = 2 * p + c
            build = make_build(q * n)
            pltpu.sync_copy(aux_ref.at[q], asl)
            pltpu.sync_copy(aux_ref.at[heads + q], adl)

            zero_acc()
            plsc.subcore_barrier()

            @pl.loop(0, nch)
            def _(ck):
                pltpu.sync_copy(src_ref.at[pl.ds(base + ck * ch, ch)], srcl)
                pltpu.sync_copy(dst_ref.at[pl.ds(base + ck * ch, ch)], dstl)
                build(0, 0)
                build(1, 1)

                @pl.loop(0, nbc // 2)
                def _(it):
                    b0 = it * 2

                    @pl.when(it > 0)
                    def _():
                        drain(0)
                    compute(b0, 0)

                    @pl.when(b0 + 2 < nbc)
                    def _():
                        build(b0 + 2, 0)

                    @pl.when(it > 0)
                    def _():
                        drain(1)
                    compute(b0 + 1, 1)

                    @pl.when(b0 + 3 < nbc)
                    def _():
                        build(b0 + 3, 1)

                drain(0)
                drain(1)

            plsc.subcore_barrier()
            flush_acc(outm_ref.at[p, c], outd_ref.at[p, c])
            plsc.subcore_barrier()

    return sc1


def _make_sc2(n, e, d2):
    np1 = _pad_rows(n)
    ept = e // (NSC * NSUB)
    ch = 2000
    nbc = ch // EB
    nch = ept // ch
    nrows = np1 // NSUB
    nseg = d2 // LANES
    ngrp = EB // LANES
    mesh = plsc.VectorSubcoreMesh(core_axis_name="c", subcore_axis_name="s")

    @functools.partial(
        pl.kernel,
        out_type=[
            jax.ShapeDtypeStruct((NSC, np1, d2), jnp.float32),
            jax.ShapeDtypeStruct((NSC, np1, LANES), jnp.float32),
        ],
        mesh=mesh,
        compiler_params=pltpu.CompilerParams(
            needs_layout_passes=False, use_tc_tiling_on_sc=False),
        scratch_types=[
            pltpu.VMEM((n,), jnp.float32),
            pltpu.VMEM((n,), jnp.float32),
            pltpu.VMEM((ch,), jnp.int32),
            pltpu.VMEM((ch,), jnp.int32),
            pltpu.VMEM((EB,), jnp.int32),
            pltpu.VMEM((EB,), jnp.int32),
            pltpu.VMEM((EB,), jnp.int32),
            pltpu.VMEM((EB,), jnp.int32),
            pltpu.VMEM((2, EB, d2), jnp.float32),
            pltpu.VMEM((2, EB, d2), jnp.float32),
            pltpu.VMEM((2, EB, LANES), jnp.float32),
            pltpu.VMEM_SHARED((np1, d2), jnp.float32),
            pltpu.VMEM_SHARED((np1, LANES), jnp.float32),
            pltpu.SemaphoreType.DMA,
            pltpu.SemaphoreType.DMA,
            pltpu.SemaphoreType.DMA,
            pltpu.SemaphoreType.DMA,
            pltpu.SemaphoreType.DMA,
            pltpu.SemaphoreType.DMA,
        ],
    )
    def sc2(h_ref, as_ref, ad_ref, src_ref, dst_ref,
            outm_ref, outd_ref,
            asl, adl, srcl, dstl, gidx0, gidx1, sidx0, sidx1,
            gbuf, msg, dbuf, accm, accd,
            gsem0, gsem1, smsem0, smsem1, sdsem0, sdsem1):
        c = lax.axis_index("c")
        s = lax.axis_index("s")
        pltpu.sync_copy(as_ref, asl)
        pltpu.sync_copy(ad_ref, adl)
        zvec = jnp.zeros((LANES,), jnp.float32)
        rchunks = [(k * EB, min(EB, nrows - k * EB))
                   for k in range((nrows + EB - 1) // EB)]
        for r in range(EB):
            for k2 in range(d2 // LANES):
                msg[0, r, pl.ds(k2 * LANES, LANES)] = zvec
            dbuf[0, r, :] = zvec
        for (r0, rn) in rchunks:
            pltpu.sync_copy(msg.at[0, pl.ds(0, rn)],
                            accm.at[pl.ds(s * nrows + r0, rn)])
            pltpu.sync_copy(dbuf.at[0, pl.ds(0, rn)],
                            accd.at[pl.ds(s * nrows + r0, rn)])
        base = (c * NSUB + s) * ept
        plsc.subcore_barrier()

        def build(b, slot):
            gidx = gidx0 if slot == 0 else gidx1
            for g in range(ngrp):
                sv = srcl[pl.ds(b * EB + g * LANES, LANES)]
                gidx[pl.ds(g * LANES, LANES)] = sv
            pltpu.async_copy(h_ref.at[gidx], gbuf.at[slot],
                             gsem0 if slot == 0 else gsem1)

        def compute(b, slot):
            off = b * EB
            gs = gsem0 if slot == 0 else gsem1
            sms = smsem0 if slot == 0 else smsem1
            sds = sdsem0 if slot == 0 else sdsem1
            gidx = gidx0 if slot == 0 else gidx1
            sidx = sidx0 if slot == 0 else sidx1
            pltpu.make_async_copy(h_ref.at[gidx], gbuf.at[slot], gs).wait()
            for g in range(ngrp):
                sv = srcl[pl.ds(off + g * LANES, LANES)]
                dv = dstl[pl.ds(off + g * LANES, LANES)]
                sidx[pl.ds(g * LANES, LANES)] = dv
                a0 = plsc.load_gather(asl, [sv]) + plsc.load_gather(adl, [dv])
                a0 = jnp.maximum(a0, 0.2 * a0)
                w0v = jnp.exp(a0)
                for j in range(LANES):
                    ei = g * LANES + j
                    w0 = w0v[j]
                    for k in range(nseg):
                        msg[slot, ei, pl.ds(k * LANES, LANES)] = (
                            gbuf[slot, ei, pl.ds(k * LANES, LANES)] * w0)
                    dbuf[slot, ei, :] = jnp.full((LANES,), w0, jnp.float32)
            pltpu.sync_copy(msg.at[slot], accm.at[sidx], add=True)
            pltpu.sync_copy(dbuf.at[slot], accd.at[sidx], add=True)

        def drain(slot):
            del slot

        @pl.loop(0, nch)
        def _(ck):
            pltpu.sync_copy(src_ref.at[pl.ds(base + ck * ch, ch)], srcl)
            pltpu.sync_copy(dst_ref.at[pl.ds(base + ck * ch, ch)], dstl)
            build(0, 0)
            build(1, 1)

            @pl.loop(0, nbc // 2)
            def _(it):
                b0 = it * 2

                @pl.when(it > 0)
                def _():
                    drain(0)
                compute(b0, 0)

                @pl.when(b0 + 2 < nbc)
                def _():
                    build(b0 + 2, 0)

                @pl.when(it > 0)
                def _():
                    drain(1)
                compute(b0 + 1, 1)

                @pl.when(b0 + 3 < nbc)
                def _():
                    build(b0 + 3, 1)

            if nbc % 2 == 1:
                drain(0)
                compute(nbc - 1, 0)
            drain(0)
            drain(1)

        plsc.subcore_barrier()
        for (r0, rn) in rchunks:
            pltpu.sync_copy(accm.at[pl.ds(s * nrows + r0, rn)],
                            msg.at[0, pl.ds(0, rn)])
            pltpu.sync_copy(msg.at[0, pl.ds(0, rn)],
                            outm_ref.at[c, pl.ds(s * nrows + r0, rn)])
            pltpu.sync_copy(accd.at[pl.ds(s * nrows + r0, rn)],
                            dbuf.at[0, pl.ds(0, rn)])
            pltpu.sync_copy(dbuf.at[0, pl.ds(0, rn)],
                            outd_ref.at[c, pl.ds(s * nrows + r0, rn)])

    return sc2


def kernel(x, edge_index, W1, a_src1, a_dst1, b1, W2, a_src2, a_dst2, b2):
    n, d_in = x.shape
    e = edge_index.shape[1]
    heads, hid = a_src1.shape
    d2 = W2.shape[1]
    np1 = _pad_rows(n)

    src = edge_index[0]
    dst = edge_index[1]

    w1r = W1.reshape(d_in, heads, hid)
    waux = jnp.concatenate([
        jnp.einsum("khc,hc->hk", w1r, a_src1),
        jnp.einsum("khc,hc->hk", w1r, a_dst1),
    ], axis=0)
    waux2 = jnp.concatenate([
        (W2 @ a_src2[0])[None, :],
        (W2 @ a_dst2[0])[None, :],
    ], axis=0)

    h_big, aux1 = _tc1(x, W1, waux, heads, hid)
    accm1, accd1 = _make_sc1(n, e, hid, heads)(h_big, aux1, src, dst)
    h2, aux2 = _tc2(accm1, accd1, W2, waux2, b1.reshape(1, -1), hid)
    accm2, accd2 = _make_sc2(n, e, d2)(
        h2, aux2[:n, 0], aux2[:n, 1], src, dst)
    out = _tc3(accm2, accd2, b2.reshape(1, -1))
    return out[:n]
